# Initial kernel scaffold; baseline (speedup 1.0000x reference)
import jax
import jax.numpy as jnp
from jax.experimental import pallas as pl


def kernel(user_feat, post_feat, parent_feat, Wu, bu, Wp, bp, Wce, bce, gce, oce, Wue, bue, Ws_pub_0, Wn_pub_0, bb_pub_0, Ws_com_0, Wn_com_0, bb_com_0, Ws_ucu_0, Wn_ucu_0, bb_ucu_0, Ws_pub_1, Wn_pub_1, bb_pub_1, Ws_com_1, Wn_com_1, bb_com_1, Ws_ucu_1, Wn_ucu_1, bb_ucu_1, g_user, o_user, g_post, o_post, Wc1, bc1, Wc2, bc2, comment_src, comment_dst, publish_src, publish_dst, ucu_src, ucu_dst):
    raise NotImplementedError("write your pallas kernel here")


# speedup vs baseline: 2.9475x; 2.9475x over previous
import functools

import jax
import jax.numpy as jnp
from jax import lax
from jax.experimental import pallas as pl
from jax.experimental.pallas import tpu as pltpu
from jax.experimental.pallas import tpu_sc as plsc

NU = 50000
NPN = 50000
DIN = 128
DE = 16
H = 128
C = 8
NQ = 4
QW = H // NQ
NSC = 2
NT = 16
CH = 128
ACC_ROWS = 50048
GARBAGE = NU
ZROWS = 136
WB = NU // NT


def _lrelu(x):
    return jnp.where(x > 0, x, 0.01 * x)


def _ln(x, g, b):
    m = jnp.mean(x, axis=-1, keepdims=True)
    d = x - m
    v = jnp.mean(d * d, axis=-1, keepdims=True)
    return d * lax.rsqrt(v + 1e-5) * g + b


def _encode_body(uf, pf, Wu, bu, Wue, bue, du_ref, pu_ref, ps_ref):
    i = pl.program_id(0)
    du_ref[...] = _lrelu(
        jnp.dot(uf[...], Wu[...], preferred_element_type=jnp.float32) + bu[...])
    pu_ref[...] = jnp.maximum(
        jnp.dot(pf[...], Wue[...], preferred_element_type=jnp.float32) + bue[...],
        0.0)

    @pl.when(i == 0)
    def _():
        ps_ref[...] = jnp.zeros_like(ps_ref)

    ps_ref[...] += jnp.sum(pf[...], axis=0, keepdims=True)


def _conv_body(parent, ps, WceA, WceB, bce, gce, oce, out_ref):
    cvec = jnp.dot(ps[...] * (1.0 / NPN), WceB[...],
                   preferred_element_type=jnp.float32) + bce[...]
    t = jnp.dot(parent[...], WceA[...], preferred_element_type=jnp.float32) + cvec
    out_ref[...] = jnp.maximum(_ln(t, gce[...], oce[...]), 0.0)


def _combine_body(du, ctx, scm, spb, cc, cp, h_ref):
    upd = (0.3 * scm[...]) / jnp.maximum(cc[...], 1.0)
    upd2 = (0.5 * spb[...]) / jnp.maximum(cp[...], 1.0)
    h_ref[...] = 0.7 * du[...] + 0.3 * (ctx[...] + upd + upd2)


def _layer_body(h, um, uc, Ws, Wn, bb, g, o, out_ref):
    umn = um[...] / jnp.maximum(uc[...], 1.0)
    y = (jnp.dot(h[...], Ws[...], preferred_element_type=jnp.float32)
         + jnp.dot(umn, Wn[...], preferred_element_type=jnp.float32) + bb[...])
    out_ref[...] = _lrelu(_ln(y, g[...], o[...]))


def _final_body(h, um, uc, Ws, Wn, bb, g, o, Wc1, bc1, Wc2, bc2, out_ref):
    umn = um[...] / jnp.maximum(uc[...], 1.0)
    y = (jnp.dot(h[...], Ws[...], preferred_element_type=jnp.float32)
         + jnp.dot(umn, Wn[...], preferred_element_type=jnp.float32) + bb[...])
    y = _ln(y, g[...], o[...])
    z = _lrelu(jnp.dot(y, Wc1[...], preferred_element_type=jnp.float32) + bc1[...])
    out_ref[...] = jnp.dot(z, Wc2[...], preferred_element_type=jnp.float32) + bc2[...]


def _rowspec(b, w):
    return pl.BlockSpec((b, w), lambda i: (i, 0))


def _fullspec(shape):
    return pl.BlockSpec(shape, lambda i: tuple(0 for _ in shape))


def _make_segsum(T2, nv):
    mesh = plsc.VectorSubcoreMesh(core_axis_name="c", subcore_axis_name="s")

    @functools.partial(
        pl.kernel,
        out_type=jax.ShapeDtypeStruct((NU, NQ, QW), jnp.float32),
        mesh=mesh,
        scratch_types=[
            pltpu.VMEM((2, 4, CH), jnp.int32),
            pltpu.VMEM((CH, QW), jnp.float32),
            pltpu.VMEM((CH, QW), jnp.float32),
            pltpu.VMEM((ZROWS, QW), jnp.float32),
            pltpu.VMEM_SHARED((ACC_ROWS, QW), jnp.float32),
            pltpu.SemaphoreType.DMA,
            pltpu.SemaphoreType.DMA,
            pltpu.SemaphoreType.DMA,
            pltpu.SemaphoreType.DMA,
        ],
        compiler_params=pltpu.CompilerParams(use_tc_tiling_on_sc=False),
    )
    def seg(vals, cidx, out, idxb, vb0, vb1, zb, acc, semi0, semi1, sem0, sem1):
        cid = lax.axis_index("c")
        sid = lax.axis_index("s")

        def zinit(i, _):
            zb[i // 2, pl.ds((i % 2) * 16, 16)] = jnp.zeros((16,), jnp.float32)
            return 0

        lax.fori_loop(0, ZROWS * 2, zinit, 0)

        for p in range(2):
            q = cid * 2 + p
            for r in range(ACC_ROWS // NT // ZROWS):
                pltpu.sync_copy(
                    zb, acc.at[pl.ds((sid * 23 + r) * ZROWS, ZROWS)])
            plsc.subcore_barrier()

            pltpu.async_copy(cidx.at[q, sid, 0], idxb.at[0], semi0)

            def slot(j, par, semA, semB, nxt):
                pltpu.make_async_copy(cidx.at[q, sid, j], idxb.at[par], semA).wait()
                pltpu.async_copy(cidx.at[q, sid, nxt], idxb.at[1 - par], semB)
                pltpu.async_copy(vals.at[idxb.at[par, 0]], vb0, sem0)
                pltpu.async_copy(vals.at[idxb.at[par, 2]], vb1, sem1)
                pltpu.make_async_copy(vals.at[idxb.at[par, 0]], vb0, sem0).wait()
                pltpu.sync_copy(vb0, acc.at[idxb.at[par, 1]], add=True)
                pltpu.make_async_copy(vals.at[idxb.at[par, 2]], vb1, sem1).wait()
                pltpu.sync_copy(vb1, acc.at[idxb.at[par, 3]], add=True)

            def step(u, _):
                j = 2 * u
                slot(j, 0, semi0, semi1, j + 1)
                slot(j + 1, 1, semi1, semi0, jnp.minimum(j + 2, T2 - 1))
                return 0

            lax.fori_loop(0, T2 // 2, step, 0)
            pltpu.make_async_copy(cidx.at[q, sid, 0], idxb.at[0], semi0).wait()
            plsc.subcore_barrier()
            pltpu.sync_copy(acc.at[pl.ds(sid * WB, WB)],
                            out.at[pl.ds(sid * WB, WB), q])
            plsc.subcore_barrier()

    return seg


CNT_ROWS = 3200


def _make_counts(K2):
    mesh = plsc.VectorSubcoreMesh(core_axis_name="c", subcore_axis_name="s")
    nseg = CNT_ROWS // NT

    @functools.partial(
        pl.kernel,
        out_type=jax.ShapeDtypeStruct((NSC, CNT_ROWS, 16), jnp.float32),
        mesh=mesh,
        scratch_types=[
            pltpu.VMEM((K2, CH), jnp.int32),
            pltpu.VMEM((CNT_ROWS, 16), jnp.float32),
            pltpu.VMEM((CNT_ROWS // CH, CH), jnp.int32),
            pltpu.VMEM_SHARED((CNT_ROWS, 16), jnp.float32),
        ],
        compiler_params=pltpu.CompilerParams(use_tc_tiling_on_sc=False,
                                             needs_layout_passes=False),
    )
    def cnt(sidx, out, sv, local, iot, acc):
        cid = lax.axis_index("c")
        sid = lax.axis_index("s")
        wid = sid * NSC + cid
        pltpu.sync_copy(sidx.at[wid], sv)

        def z(i, _):
            local[i, pl.ds(0, 16)] = jnp.zeros((16,), jnp.float32)
            return 0

        lax.fori_loop(0, CNT_ROWS, z, 0)
        lane = lax.iota(jnp.int32, 16)

        def zi(i, _):
            iot[i // 8, pl.ds((i % 8) * 16, 16)] = i * 16 + lane
            return 0

        lax.fori_loop(0, CNT_ROWS // 16, zi, 0)
        pltpu.sync_copy(local.at[pl.ds(sid * nseg, nseg)],
                        acc.at[pl.ds(sid * nseg, nseg)])
        plsc.subcore_barrier()
        ones = jnp.ones((16,), jnp.float32)

        def step(j, _):
            for cb in range(CH // 16):
                v = sv[j, pl.ds(cb * 16, 16)]
                plsc.addupdate_scatter(
                    local,
                    [lax.shift_right_logical(v, 4), lax.bitwise_and(v, 15)],
                    ones)
            return 0

        lax.fori_loop(0, K2, step, 0)
        for t in range(CNT_ROWS // CH):
            pltpu.sync_copy(local.at[pl.ds(t * CH, CH)], acc.at[iot.at[t]],
                            add=True)
        plsc.subcore_barrier()
        pltpu.sync_copy(acc.at[pl.ds(sid * nseg, nseg)],
                        out.at[cid].at[pl.ds(sid * nseg, nseg)])

    return cnt


def _prep_seg_idx(g, s, e, nv):
    k = -(-e // (NT * CH))
    k = (k + 3) // 4 * 4
    pad = NT * k * CH - e
    g = g.astype(jnp.int32)
    s = s.astype(jnp.int32)
    if pad:
        g = jnp.concatenate([g, jnp.zeros((pad,), jnp.int32)])
        s = jnp.concatenate([s, jnp.full((pad,), GARBAGE, jnp.int32)])
    g4 = (g[None] * NQ + jnp.arange(NQ, dtype=jnp.int32)[:, None])
    g4 = g4.reshape(NQ, NT, k // 2, 2, CH)
    s4 = jnp.broadcast_to(s.reshape(1, NT, k // 2, 2, CH),
                          (NQ, NT, k // 2, 2, CH))
    comb = jnp.stack([g4[..., 0, :], s4[..., 0, :],
                      g4[..., 1, :], s4[..., 1, :]], axis=3)
    return comb, k // 2


def _prep_cnt_idx(s, e):
    k2 = -(-e // (NT * NSC * CH))
    pad = NT * NSC * k2 * CH - e
    s = s.astype(jnp.int32)
    if pad:
        s = jnp.concatenate([s, jnp.full((pad,), GARBAGE, jnp.int32)])
    return s.reshape(NT * NSC, k2, CH), k2


def _counts(sidx, e):
    s3, k2 = _prep_cnt_idx(sidx, e)
    part = _make_counts(k2)(s3)
    return (part[0] + part[1]).reshape(CNT_ROWS * 16)[:NU].reshape(NU, 1)


def _segsum(vals, gidx, sidx, e, nv):
    comb, t2 = _prep_seg_idx(gidx, sidx, e, nv)
    out = _make_segsum(t2, nv)(vals.reshape(NQ * nv, QW), comb)
    return out.reshape(NU, H)


def kernel(user_feat, post_feat, parent_feat, Wu, bu, Wp, bp, Wce, bce, gce, oce,
           Wue, bue, Ws_pub_0, Wn_pub_0, bb_pub_0, Ws_com_0, Wn_com_0, bb_com_0,
           Ws_ucu_0, Wn_ucu_0, bb_ucu_0, Ws_pub_1, Wn_pub_1, bb_pub_1, Ws_com_1,
           Wn_com_1, bb_com_1, Ws_ucu_1, Wn_ucu_1, bb_ucu_1, g_user, o_user,
           g_post, o_post, Wc1, bc1, Wc2, bc2, comment_src, comment_dst,
           publish_src, publish_dst, ucu_src, ucu_dst):
    f32 = jnp.float32
    EC = comment_src.shape[0]
    EP = publish_src.shape[0]
    EU = ucu_src.shape[0]
    r1 = lambda v: v.reshape(1, -1)

    cnt_c = _counts(comment_src, EC)
    cnt_p = _counts(publish_src, EP)
    ucnt = _counts(ucu_dst, EU)

    B1, G1 = 400, NU // 400
    du, pu, psum = pl.pallas_call(
        _encode_body,
        grid=(G1,),
        in_specs=[_rowspec(B1, DIN), _rowspec(B1, DIN), _fullspec((DIN, H)),
                  _fullspec((1, H)), _fullspec((DIN, H)), _fullspec((1, H))],
        out_specs=[_rowspec(B1, H), _rowspec(B1, H), _fullspec((1, H))],
        out_shape=[jax.ShapeDtypeStruct((NU, H), f32),
                   jax.ShapeDtypeStruct((NPN, H), f32),
                   jax.ShapeDtypeStruct((1, H), f32)],
    )(user_feat, post_feat, Wu, r1(bu), Wue, r1(bue))

    B2, G2 = 1000, EU // 1000
    conv = pl.pallas_call(
        _conv_body,
        grid=(G2,),
        in_specs=[_rowspec(B2, DE), _fullspec((1, H)), _fullspec((DE, H)),
                  _fullspec((DIN, H)), _fullspec((1, H)), _fullspec((1, H)),
                  _fullspec((1, H))],
        out_specs=_rowspec(B2, H),
        out_shape=jax.ShapeDtypeStruct((EU, H), f32),
    )(parent_feat, psum, Wce[:DE], Wce[DE:], r1(bce), r1(gce), r1(oce))

    s_com = _segsum(pu, comment_dst, comment_src, EC, NPN)
    s_pub = _segsum(pu, publish_dst, publish_src, EP, NPN)
    ctx = _segsum(conv, jnp.arange(EU, dtype=jnp.int32), ucu_src, EU, EU)

    B3, G3 = 400, NU // 400
    h = pl.pallas_call(
        _combine_body,
        grid=(G3,),
        in_specs=[_rowspec(B3, H)] * 4 + [_rowspec(B3, 1)] * 2,
        out_specs=_rowspec(B3, H),
        out_shape=jax.ShapeDtypeStruct((NU, H), f32),
    )(du, ctx, s_com, s_pub, cnt_c, cnt_p)

    um0 = _segsum(h, ucu_src, ucu_dst, EU, NU)
    h1 = pl.pallas_call(
        _layer_body,
        grid=(G3,),
        in_specs=[_rowspec(B3, H), _rowspec(B3, H), _rowspec(B3, 1),
                  _fullspec((H, H)), _fullspec((H, H)), _fullspec((1, H)),
                  _fullspec((1, H)), _fullspec((1, H))],
        out_specs=_rowspec(B3, H),
        out_shape=jax.ShapeDtypeStruct((NU, H), f32),
    )(h, um0, ucnt, Ws_ucu_0, Wn_ucu_0, r1(bb_ucu_0), r1(g_user), r1(o_user))

    um1 = _segsum(h1, ucu_src, ucu_dst, EU, NU)
    out = pl.pallas_call(
        _final_body,
        grid=(G3,),
        in_specs=[_rowspec(B3, H), _rowspec(B3, H), _rowspec(B3, 1),
                  _fullspec((H, H)), _fullspec((H, H)), _fullspec((1, H)),
                  _fullspec((1, H)), _fullspec((1, H)), _fullspec((H, H)),
                  _fullspec((1, H)), _fullspec((H, C)), _fullspec((1, C))],
        out_specs=_rowspec(B3, C),
        out_shape=jax.ShapeDtypeStruct((NU, C), f32),
    )(h1, um1, ucnt, Ws_ucu_1, Wn_ucu_1, r1(bb_ucu_1), r1(g_user), r1(o_user),
      Wc1, r1(bc1), Wc2, r1(bc2))
    return out


# speedup vs baseline: 3.1180x; 1.0578x over previous
import functools

import jax
import jax.numpy as jnp
from jax import lax
from jax.experimental import pallas as pl
from jax.experimental.pallas import tpu as pltpu
from jax.experimental.pallas import tpu_sc as plsc

NU = 50000
NPN = 50000
DIN = 128
DE = 16
H = 128
C = 8
NQ = 4
QW = H // NQ
NSC = 2
NT = 16
CH = 128
ACC_ROWS = 50048
GARBAGE = NU
ZROWS = 136
WB = NU // NT


def _lrelu(x):
    return jnp.where(x > 0, x, 0.01 * x)


def _ln(x, g, b):
    m = jnp.mean(x, axis=-1, keepdims=True)
    d = x - m
    v = jnp.mean(d * d, axis=-1, keepdims=True)
    return d * lax.rsqrt(v + 1e-5) * g + b


def _encode_body(uf, pf, Wu, bu, Wue, bue, du_ref, pu_ref, ps_ref):
    i = pl.program_id(0)
    du_ref[...] = _lrelu(
        jnp.dot(uf[...], Wu[...], preferred_element_type=jnp.float32) + bu[...])
    pu_ref[...] = jnp.maximum(
        jnp.dot(pf[...], Wue[...], preferred_element_type=jnp.float32) + bue[...],
        0.0)

    @pl.when(i == 0)
    def _():
        ps_ref[...] = jnp.zeros_like(ps_ref)

    ps_ref[...] += jnp.sum(pf[...], axis=0, keepdims=True)


def _conv_body(parent, ps, WceA, WceB, bce, gce, oce, out_ref):
    cvec = jnp.dot(ps[...] * (1.0 / NPN), WceB[...],
                   preferred_element_type=jnp.float32) + bce[...]
    t = jnp.dot(parent[...], WceA[...], preferred_element_type=jnp.float32) + cvec
    out_ref[...] = jnp.maximum(_ln(t, gce[...], oce[...]), 0.0)


def _combine_body(du, ctx, scm, spb, cc, cp, h_ref):
    upd = (0.3 * scm[...]) / jnp.maximum(cc[...], 1.0)
    upd2 = (0.5 * spb[...]) / jnp.maximum(cp[...], 1.0)
    h_ref[...] = 0.7 * du[...] + 0.3 * (ctx[...] + upd + upd2)


def _layer_body(h, um, uc, Ws, Wn, bb, g, o, out_ref):
    umn = um[...] / jnp.maximum(uc[...], 1.0)
    y = (jnp.dot(h[...], Ws[...], preferred_element_type=jnp.float32)
         + jnp.dot(umn, Wn[...], preferred_element_type=jnp.float32) + bb[...])
    out_ref[...] = _lrelu(_ln(y, g[...], o[...]))


def _final_body(h, um, uc, Ws, Wn, bb, g, o, Wc1, bc1, Wc2, bc2, out_ref):
    umn = um[...] / jnp.maximum(uc[...], 1.0)
    y = (jnp.dot(h[...], Ws[...], preferred_element_type=jnp.float32)
         + jnp.dot(umn, Wn[...], preferred_element_type=jnp.float32) + bb[...])
    y = _ln(y, g[...], o[...])
    z = _lrelu(jnp.dot(y, Wc1[...], preferred_element_type=jnp.float32) + bc1[...])
    out_ref[...] = jnp.dot(z, Wc2[...], preferred_element_type=jnp.float32) + bc2[...]


def _rowspec(b, w):
    return pl.BlockSpec((b, w), lambda i: (i, 0))


def _fullspec(shape):
    return pl.BlockSpec(shape, lambda i: tuple(0 for _ in shape))


def _make_segsum(T2, nv):
    mesh = plsc.VectorSubcoreMesh(core_axis_name="c", subcore_axis_name="s")

    @functools.partial(
        pl.kernel,
        out_type=jax.ShapeDtypeStruct((NU, NQ, QW), jnp.float32),
        mesh=mesh,
        scratch_types=[
            pltpu.VMEM((2, 4, CH), jnp.int32),
            pltpu.VMEM((CH, QW), jnp.float32),
            pltpu.VMEM((CH, QW), jnp.float32),
            pltpu.VMEM((CH, QW), jnp.float32),
            pltpu.VMEM((CH, QW), jnp.float32),
            pltpu.VMEM((ZROWS, QW), jnp.float32),
            pltpu.VMEM_SHARED((ACC_ROWS, QW), jnp.float32),
            pltpu.SemaphoreType.DMA,
            pltpu.SemaphoreType.DMA,
            pltpu.SemaphoreType.DMA,
            pltpu.SemaphoreType.DMA,
            pltpu.SemaphoreType.DMA,
            pltpu.SemaphoreType.DMA,
        ],
        compiler_params=pltpu.CompilerParams(use_tc_tiling_on_sc=False),
    )
    def seg(vals, cidx, out, idxb, vb00, vb01, vb10, vb11, zb, acc,
            isem0, isem1, vsem00, vsem01, vsem10, vsem11):
        cid = lax.axis_index("c")
        sid = lax.axis_index("s")

        def zinit(i, _):
            zb[i // 2, pl.ds((i % 2) * 16, 16)] = jnp.zeros((16,), jnp.float32)
            return 0

        lax.fori_loop(0, ZROWS * 2, zinit, 0)

        for p in range(2):
            q = cid * 2 + p
            for r in range(ACC_ROWS // NT // ZROWS):
                pltpu.sync_copy(
                    zb, acc.at[pl.ds((sid * 23 + r) * ZROWS, ZROWS)])
            plsc.subcore_barrier()

            pltpu.async_copy(cidx.at[q, sid, 0], idxb.at[0], isem0)
            pltpu.make_async_copy(cidx.at[q, sid, 0], idxb.at[0], isem0).wait()
            pltpu.async_copy(vals.at[idxb.at[0, 0]], vb00, vsem00)
            pltpu.async_copy(vals.at[idxb.at[0, 2]], vb01, vsem01)
            pltpu.async_copy(cidx.at[q, sid, jnp.minimum(1, T2 - 1)],
                             idxb.at[1], isem1)

            def slot(j, par, isemA, isemB, vbA0, vbA1, vsA0, vsA1,
                     vbB0, vbB1, vsB0, vsB1):
                pltpu.make_async_copy(
                    cidx.at[q, sid, j], idxb.at[1 - par], isemB).wait()
                pltpu.async_copy(vals.at[idxb.at[1 - par, 0]], vbB0, vsB0)
                pltpu.async_copy(vals.at[idxb.at[1 - par, 2]], vbB1, vsB1)
                pltpu.make_async_copy(vals.at[idxb.at[par, 0]], vbA0, vsA0).wait()
                pltpu.sync_copy(vbA0, acc.at[idxb.at[par, 1]], add=True)
                pltpu.make_async_copy(vals.at[idxb.at[par, 2]], vbA1, vsA1).wait()
                pltpu.sync_copy(vbA1, acc.at[idxb.at[par, 3]], add=True)
                pltpu.async_copy(cidx.at[q, sid, jnp.minimum(j + 1, T2 - 1)],
                                 idxb.at[par], isemA)

            def step(u, _):
                j = 2 * u
                slot(j + 1, 0, isem0, isem1,
                     vb00, vb01, vsem00, vsem01, vb10, vb11, vsem10, vsem11)
                slot(jnp.minimum(j + 2, T2 - 1), 1, isem1, isem0,
                     vb10, vb11, vsem10, vsem11, vb00, vb01, vsem00, vsem01)
                return 0

            lax.fori_loop(0, T2 // 2, step, 0)
            pltpu.make_async_copy(cidx.at[q, sid, 0], idxb.at[1], isem1).wait()
            pltpu.make_async_copy(vals.at[idxb.at[0, 0]], vb00, vsem00).wait()
            pltpu.make_async_copy(vals.at[idxb.at[0, 2]], vb01, vsem01).wait()
            plsc.subcore_barrier()
            pltpu.sync_copy(acc.at[pl.ds(sid * WB, WB)],
                            out.at[pl.ds(sid * WB, WB), q])
            plsc.subcore_barrier()

    return seg


CNT_ROWS = 3200


def _make_counts(K2):
    mesh = plsc.VectorSubcoreMesh(core_axis_name="c", subcore_axis_name="s")
    nseg = CNT_ROWS // NT

    @functools.partial(
        pl.kernel,
        out_type=jax.ShapeDtypeStruct((NSC, CNT_ROWS, 16), jnp.float32),
        mesh=mesh,
        scratch_types=[
            pltpu.VMEM((K2, CH), jnp.int32),
            pltpu.VMEM((CNT_ROWS, 16), jnp.float32),
            pltpu.VMEM((CNT_ROWS // CH, CH), jnp.int32),
            pltpu.VMEM_SHARED((CNT_ROWS, 16), jnp.float32),
        ],
        compiler_params=pltpu.CompilerParams(use_tc_tiling_on_sc=False,
                                             needs_layout_passes=False),
    )
    def cnt(sidx, out, sv, local, iot, acc):
        cid = lax.axis_index("c")
        sid = lax.axis_index("s")
        wid = sid * NSC + cid
        pltpu.sync_copy(sidx.at[wid], sv)

        def z(i, _):
            local[i, pl.ds(0, 16)] = jnp.zeros((16,), jnp.float32)
            return 0

        lax.fori_loop(0, CNT_ROWS, z, 0)
        lane = lax.iota(jnp.int32, 16)

        def zi(i, _):
            iot[i // 8, pl.ds((i % 8) * 16, 16)] = i * 16 + lane
            return 0

        lax.fori_loop(0, CNT_ROWS // 16, zi, 0)
        pltpu.sync_copy(local.at[pl.ds(sid * nseg, nseg)],
                        acc.at[pl.ds(sid * nseg, nseg)])
        plsc.subcore_barrier()
        ones = jnp.ones((16,), jnp.float32)

        def step(j, _):
            for cb in range(CH // 16):
                v = sv[j, pl.ds(cb * 16, 16)]
                plsc.addupdate_scatter(
                    local,
                    [lax.shift_right_logical(v, 4), lax.bitwise_and(v, 15)],
                    ones)
            return 0

        lax.fori_loop(0, K2, step, 0)
        for t in range(CNT_ROWS // CH):
            pltpu.sync_copy(local.at[pl.ds(t * CH, CH)], acc.at[iot.at[t]],
                            add=True)
        plsc.subcore_barrier()
        pltpu.sync_copy(acc.at[pl.ds(sid * nseg, nseg)],
                        out.at[cid].at[pl.ds(sid * nseg, nseg)])

    return cnt


def _prep_seg_idx(g, s, e, nv):
    k = -(-e // (NT * CH))
    k = (k + 3) // 4 * 4
    pad = NT * k * CH - e
    g = g.astype(jnp.int32)
    s = s.astype(jnp.int32)
    if pad:
        g = jnp.concatenate([g, jnp.zeros((pad,), jnp.int32)])
        s = jnp.concatenate([s, jnp.full((pad,), GARBAGE, jnp.int32)])
    g4 = (g[None] * NQ + jnp.arange(NQ, dtype=jnp.int32)[:, None])
    g4 = g4.reshape(NQ, NT, k // 2, 2, CH)
    s4 = jnp.broadcast_to(s.reshape(1, NT, k // 2, 2, CH),
                          (NQ, NT, k // 2, 2, CH))
    comb = jnp.stack([g4[..., 0, :], s4[..., 0, :],
                      g4[..., 1, :], s4[..., 1, :]], axis=3)
    return comb, k // 2


def _prep_cnt_idx(s, e):
    k2 = -(-e // (NT * NSC * CH))
    pad = NT * NSC * k2 * CH - e
    s = s.astype(jnp.int32)
    if pad:
        s = jnp.concatenate([s, jnp.full((pad,), GARBAGE, jnp.int32)])
    return s.reshape(NT * NSC, k2, CH), k2


def _counts(sidx, e):
    s3, k2 = _prep_cnt_idx(sidx, e)
    part = _make_counts(k2)(s3)
    return (part[0] + part[1]).reshape(CNT_ROWS * 16)[:NU].reshape(NU, 1)


def _segsum(vals, gidx, sidx, e, nv):
    comb, t2 = _prep_seg_idx(gidx, sidx, e, nv)
    out = _make_segsum(t2, nv)(vals.reshape(NQ * nv, QW), comb)
    return out.reshape(NU, H)


def kernel(user_feat, post_feat, parent_feat, Wu, bu, Wp, bp, Wce, bce, gce, oce,
           Wue, bue, Ws_pub_0, Wn_pub_0, bb_pub_0, Ws_com_0, Wn_com_0, bb_com_0,
           Ws_ucu_0, Wn_ucu_0, bb_ucu_0, Ws_pub_1, Wn_pub_1, bb_pub_1, Ws_com_1,
           Wn_com_1, bb_com_1, Ws_ucu_1, Wn_ucu_1, bb_ucu_1, g_user, o_user,
           g_post, o_post, Wc1, bc1, Wc2, bc2, comment_src, comment_dst,
           publish_src, publish_dst, ucu_src, ucu_dst):
    f32 = jnp.float32
    EC = comment_src.shape[0]
    EP = publish_src.shape[0]
    EU = ucu_src.shape[0]
    r1 = lambda v: v.reshape(1, -1)

    cnt_c = _counts(comment_src, EC)
    cnt_p = _counts(publish_src, EP)
    ucnt = _counts(ucu_dst, EU)

    B1, G1 = 400, NU // 400
    du, pu, psum = pl.pallas_call(
        _encode_body,
        grid=(G1,),
        in_specs=[_rowspec(B1, DIN), _rowspec(B1, DIN), _fullspec((DIN, H)),
                  _fullspec((1, H)), _fullspec((DIN, H)), _fullspec((1, H))],
        out_specs=[_rowspec(B1, H), _rowspec(B1, H), _fullspec((1, H))],
        out_shape=[jax.ShapeDtypeStruct((NU, H), f32),
                   jax.ShapeDtypeStruct((NPN, H), f32),
                   jax.ShapeDtypeStruct((1, H), f32)],
    )(user_feat, post_feat, Wu, r1(bu), Wue, r1(bue))

    B2, G2 = 1000, EU // 1000
    conv = pl.pallas_call(
        _conv_body,
        grid=(G2,),
        in_specs=[_rowspec(B2, DE), _fullspec((1, H)), _fullspec((DE, H)),
                  _fullspec((DIN, H)), _fullspec((1, H)), _fullspec((1, H)),
                  _fullspec((1, H))],
        out_specs=_rowspec(B2, H),
        out_shape=jax.ShapeDtypeStruct((EU, H), f32),
    )(parent_feat, psum, Wce[:DE], Wce[DE:], r1(bce), r1(gce), r1(oce))

    s_com = _segsum(pu, comment_dst, comment_src, EC, NPN)
    s_pub = _segsum(pu, publish_dst, publish_src, EP, NPN)
    ctx = _segsum(conv, jnp.arange(EU, dtype=jnp.int32), ucu_src, EU, EU)

    B3, G3 = 400, NU // 400
    h = pl.pallas_call(
        _combine_body,
        grid=(G3,),
        in_specs=[_rowspec(B3, H)] * 4 + [_rowspec(B3, 1)] * 2,
        out_specs=_rowspec(B3, H),
        out_shape=jax.ShapeDtypeStruct((NU, H), f32),
    )(du, ctx, s_com, s_pub, cnt_c, cnt_p)

    um0 = _segsum(h, ucu_src, ucu_dst, EU, NU)
    h1 = pl.pallas_call(
        _layer_body,
        grid=(G3,),
        in_specs=[_rowspec(B3, H), _rowspec(B3, H), _rowspec(B3, 1),
                  _fullspec((H, H)), _fullspec((H, H)), _fullspec((1, H)),
                  _fullspec((1, H)), _fullspec((1, H))],
        out_specs=_rowspec(B3, H),
        out_shape=jax.ShapeDtypeStruct((NU, H), f32),
    )(h, um0, ucnt, Ws_ucu_0, Wn_ucu_0, r1(bb_ucu_0), r1(g_user), r1(o_user))

    um1 = _segsum(h1, ucu_src, ucu_dst, EU, NU)
    out = pl.pallas_call(
        _final_body,
        grid=(G3,),
        in_specs=[_rowspec(B3, H), _rowspec(B3, H), _rowspec(B3, 1),
                  _fullspec((H, H)), _fullspec((H, H)), _fullspec((1, H)),
                  _fullspec((1, H)), _fullspec((1, H)), _fullspec((H, H)),
                  _fullspec((1, H)), _fullspec((H, C)), _fullspec((1, C))],
        out_specs=_rowspec(B3, C),
        out_shape=jax.ShapeDtypeStruct((NU, C), f32),
    )(h1, um1, ucnt, Ws_ucu_1, Wn_ucu_1, r1(bb_ucu_1), r1(g_user), r1(o_user),
      Wc1, r1(bc1), Wc2, r1(bc2))
    return out
